# baseline (speedup 1.0000x reference)
import functools
import jax
import jax.numpy as jnp
from jax import lax
from jax.experimental import pallas as pl
from jax.experimental.pallas import tpu as pltpu

_TE = 2000


def _edge_dense_body(ea_ref, meo_ref, ms_ref, mr_ref, we0_ref, w1c_ref,
                     w3b_ref, w2m_ref, consts_ref, eo_ref, f_ref, lg_ref):
    c0 = consts_ref[0, :]
    b1 = consts_ref[1, :]
    ea = ea_ref[...]
    eo = jnp.dot(ea, we0_ref[...], preferred_element_type=jnp.float32)
    eo = eo + meo_ref[...] + c0[None, :]
    eo_ref[...] = eo
    C = jnp.dot(eo, w1c_ref[...], preferred_element_type=jnp.float32)
    f_ref[...] = jnp.dot(eo, w3b_ref[...], preferred_element_type=jnp.float32)
    xs = ms_ref[...] + C + b1[None, :]
    xr = mr_ref[...] + C + b1[None, :]
    xs = jnp.where(xs > 0, xs, 0.01 * xs)
    xr = jnp.where(xr > 0, xr, 0.01 * xr)
    w2m = w2m_ref[...]
    ls = jnp.dot(xs, w2m, preferred_element_type=jnp.float32)
    lr = jnp.dot(xr, w2m, preferred_element_type=jnp.float32)
    lg_ref[...] = jnp.concatenate([ls[:, :2], lr[:, :2], ls[:, :2], lr[:, :2]],
                                  axis=1)


def _edge_dense(edge_attr, m_eo, m_s, m_r, We0, w1c, w3b, W2m, consts):
    E, D = edge_attr.shape
    grid = E // _TE
    blk = lambda i: (i, 0)
    whole = lambda i: (0, 0)
    return pl.pallas_call(
        _edge_dense_body,
        grid=(grid,),
        in_specs=[
            pl.BlockSpec((_TE, D), blk),
            pl.BlockSpec((_TE, D), blk),
            pl.BlockSpec((_TE, D), blk),
            pl.BlockSpec((_TE, D), blk),
            pl.BlockSpec((D, D), whole),
            pl.BlockSpec((D, D), whole),
            pl.BlockSpec((D, D), whole),
            pl.BlockSpec((D, 8), whole),
            pl.BlockSpec((8, D), whole),
        ],
        out_specs=[
            pl.BlockSpec((_TE, D), blk),
            pl.BlockSpec((_TE, D), blk),
            pl.BlockSpec((_TE, 8), blk),
        ],
        out_shape=[
            jax.ShapeDtypeStruct((E, D), jnp.float32),
            jax.ShapeDtypeStruct((E, D), jnp.float32),
            jax.ShapeDtypeStruct((E, 8), jnp.float32),
        ],
    )(edge_attr, m_eo, m_s, m_r, We0, w1c, w3b, W2m, consts)


def kernel(node_attr, edge_attr, global_attr, edge_index, node_batch,
           edge_batch, num_nodes, num_edges, W_edge, b_edge, W_node, b_node,
           w1, b1, w2, w3, b3):
    N, D = node_attr.shape
    E = edge_attr.shape[0]
    H, HD = w2.shape
    row = edge_index[0]
    col = edge_index[1]

    A_e = node_attr @ W_edge[D:2 * D]
    B_e = edge_attr[:N] @ W_edge[2 * D:3 * D]
    c0 = global_attr @ W_edge[3 * D:] + b_edge
    A1 = node_attr @ w1[:D]
    B1 = node_attr @ w1[D:2 * D]
    Dv = node_attr @ w3[:D]
    cn = global_attr @ W_node[3 * D:] + b_node

    W2m = jnp.zeros((D, 8), jnp.float32)
    for h in range(H):
        W2m = W2m.at[h * HD:(h + 1) * HD, h].set(w2[h])

    consts = jnp.zeros((8, D), jnp.float32)
    consts = consts.at[0].set(c0[0]).at[1].set(b1)

    m_eo = A_e[row] + B_e[col]
    m_s = A1[row] + B1[col]
    m_r = B1[row] + A1[col]
    dv_c = Dv[col]
    dv_r = Dv[row]

    edge_out, F, lg = _edge_dense(edge_attr, m_eo, m_s, m_r, W_edge[:D],
                                  w1[2 * D:], w3[D:], W2m, consts)
    ls = lg[:, 0:2]
    lr = lg[:, 2:4]

    max_s = jax.ops.segment_max(ls, row, N)
    max_r = jax.ops.segment_max(lr, col, N)
    max_s = jnp.where(jnp.isfinite(max_s), max_s, 0.0)
    max_r = jnp.where(jnp.isfinite(max_r), max_r, 0.0)
    ex_s = jnp.exp(ls - max_s[row])
    ex_r = jnp.exp(lr - max_r[col])

    v_s = (dv_c + F + b3[None, :]).reshape(E, H, HD)
    v_r = (dv_r + F + b3[None, :]).reshape(E, H, HD)
    u_s = (ex_s[:, :, None] * v_s).reshape(E, D)
    u_r = (ex_r[:, :, None] * v_r).reshape(E, D)

    sent_raw = jax.ops.segment_sum(u_s, row, N)
    recv_raw = jax.ops.segment_sum(u_r, col, N)
    den_s = jax.ops.segment_sum(ex_s, row, N)
    den_r = jax.ops.segment_sum(ex_r, col, N)

    sent = (sent_raw.reshape(N, H, HD) /
            (den_s[:, :, None] + 1e-16)).reshape(N, D)
    recv = (recv_raw.reshape(N, H, HD) /
            (den_r[:, :, None] + 1e-16)).reshape(N, D)

    node_out = (node_attr @ W_node[:D] + sent @ W_node[D:2 * D] +
                recv @ W_node[2 * D:3 * D] + cn)
    return (node_out, edge_out, global_attr)


# speedup vs baseline: 1.0197x; 1.0197x over previous
import jax
import jax.numpy as jnp
from jax import lax
from jax.experimental import pallas as pl
from jax.experimental.pallas import tpu as pltpu
from jax.experimental.pallas import tpu_sc as plsc

_TE = 2000
_CH = 128
_NEG = -3.0e38


def _segmax_body(lg_hbm, row_hbm, col_hbm, out_hbm,
                 lgbuf, idxbuf, lgtail, idxtail, mt0, mt1, mt2, mt3):
    E = lg_hbm.shape[0] // 8
    N = mt0.shape[0]
    c = lax.axis_index("c")
    s = lax.axis_index("s")
    w = s * 2 + c
    mts = (mt0, mt1, mt2, mt3)

    def _init(i, _):
        mt0[pl.ds(i * 16, 16)] = jnp.full((16,), _NEG, jnp.float32)
        mt1[pl.ds(i * 16, 16)] = jnp.full((16,), _NEG, jnp.float32)
        mt2[pl.ds(i * 16, 16)] = jnp.full((16,), _NEG, jnp.float32)
        mt3[pl.ds(i * 16, 16)] = jnp.full((16,), _NEG, jnp.float32)
        return 0
    lax.fori_loop(0, N // 16, _init, 0)

    lanes = lax.iota(jnp.int32, 16)

    def _scatter_max(mt, idxv, val):
        cur = plsc.load_gather(mt, [idxv])
        tgt = jnp.maximum(cur, val)
        plsc.store_scatter(mt, [idxv], tgt)

        def cond(st):
            return st[0]

        def body(st):
            _, t = st
            cur2 = plsc.load_gather(mt, [idxv])
            need = cur2 < t
            plsc.store_scatter(mt, [idxv], jnp.maximum(cur2, t), mask=need)
            cur3 = plsc.load_gather(mt, [idxv])
            return (jnp.any(cur3 < t), t)

        first = plsc.load_gather(mt, [idxv])
        lax.while_loop(cond, body, (jnp.any(first < tgt), tgt))

    def _fields(lbuf, ibuf, g, fs):
        ids8 = (lanes + g * 16) * 8
        idxv = ibuf[pl.ds(g * 16, 16)]
        for f in fs:
            val = plsc.load_gather(lbuf, [ids8 + f])
            _scatter_max(mts[f], idxv, val)

    per_tile = (E // 32) // 16 * 16
    n_chunks = per_tile // _CH

    def _chunk(k, _):
        base = w * per_tile + k * _CH

        def _grp_row(g, _):
            _fields(lgbuf, idxbuf, g, (0, 1))
            return 0

        def _grp_col(g, _):
            _fields(lgbuf, idxbuf, g, (2, 3))
            return 0

        pltpu.sync_copy(lg_hbm.at[pl.ds(base * 8, _CH * 8)], lgbuf)
        pltpu.sync_copy(row_hbm.at[pl.ds(base, _CH)], idxbuf)
        lax.fori_loop(0, _CH // 16, _grp_row, 0)
        pltpu.sync_copy(col_hbm.at[pl.ds(base, _CH)], idxbuf)
        lax.fori_loop(0, _CH // 16, _grp_col, 0)
        return 0
    lax.fori_loop(0, n_chunks, _chunk, 0)

    rem = E - 32 * per_tile
    n_rem_groups = rem // 16
    if rem:
        @pl.when(w < n_rem_groups)
        def _():
            base = 32 * per_tile + w * 16
            pltpu.sync_copy(lg_hbm.at[pl.ds(base * 8, 128)], lgtail)
            pltpu.sync_copy(row_hbm.at[pl.ds(base, 16)], idxtail)
            _fields(lgtail, idxtail, 0, (0, 1))
            pltpu.sync_copy(col_hbm.at[pl.ds(base, 16)], idxtail)
            _fields(lgtail, idxtail, 0, (2, 3))

    pltpu.sync_copy(mt0, out_hbm.at[w, 0])
    pltpu.sync_copy(mt1, out_hbm.at[w, 1])
    pltpu.sync_copy(mt2, out_hbm.at[w, 2])
    pltpu.sync_copy(mt3, out_hbm.at[w, 3])


def _sc_segmax(lg, row, col, N):
    mesh = plsc.VectorSubcoreMesh(core_axis_name="c", subcore_axis_name="s")
    f = pl.kernel(
        _segmax_body,
        out_type=jax.ShapeDtypeStruct((32, 4, N), jnp.float32),
        mesh=mesh,
        compiler_params=pltpu.CompilerParams(needs_layout_passes=False),
        scratch_types=[
            pltpu.VMEM((_CH * 8,), jnp.float32),
            pltpu.VMEM((_CH,), jnp.int32),
            pltpu.VMEM((128,), jnp.float32),
            pltpu.VMEM((16,), jnp.int32),
            pltpu.VMEM((N,), jnp.float32),
            pltpu.VMEM((N,), jnp.float32),
            pltpu.VMEM((N,), jnp.float32),
            pltpu.VMEM((N,), jnp.float32),
        ],
    )
    return f(lg.reshape(-1), row, col)


def _edge_dense_body(ea_ref, meo_ref, ms_ref, mr_ref, we0_ref, w1c_ref,
                     w3b_ref, w2m_ref, consts_ref, eo_ref, f_ref, lg_ref):
    c0 = consts_ref[0, :]
    b1 = consts_ref[1, :]
    ea = ea_ref[...]
    eo = jnp.dot(ea, we0_ref[...], preferred_element_type=jnp.float32)
    eo = eo + meo_ref[...] + c0[None, :]
    eo_ref[...] = eo
    C = jnp.dot(eo, w1c_ref[...], preferred_element_type=jnp.float32)
    f_ref[...] = jnp.dot(eo, w3b_ref[...], preferred_element_type=jnp.float32)
    xs = ms_ref[...] + C + b1[None, :]
    xr = mr_ref[...] + C + b1[None, :]
    xs = jnp.where(xs > 0, xs, 0.01 * xs)
    xr = jnp.where(xr > 0, xr, 0.01 * xr)
    w2m = w2m_ref[...]
    ls = jnp.dot(xs, w2m, preferred_element_type=jnp.float32)
    lr = jnp.dot(xr, w2m, preferred_element_type=jnp.float32)
    lg_ref[...] = jnp.concatenate([ls[:, :2], lr[:, :2], ls[:, :2], lr[:, :2]],
                                  axis=1)


def _edge_dense(edge_attr, m_eo, m_s, m_r, We0, w1c, w3b, W2m, consts):
    E, D = edge_attr.shape
    grid = E // _TE
    blk = lambda i: (i, 0)
    whole = lambda i: (0, 0)
    return pl.pallas_call(
        _edge_dense_body,
        grid=(grid,),
        in_specs=[
            pl.BlockSpec((_TE, D), blk),
            pl.BlockSpec((_TE, D), blk),
            pl.BlockSpec((_TE, D), blk),
            pl.BlockSpec((_TE, D), blk),
            pl.BlockSpec((D, D), whole),
            pl.BlockSpec((D, D), whole),
            pl.BlockSpec((D, D), whole),
            pl.BlockSpec((D, 8), whole),
            pl.BlockSpec((8, D), whole),
        ],
        out_specs=[
            pl.BlockSpec((_TE, D), blk),
            pl.BlockSpec((_TE, D), blk),
            pl.BlockSpec((_TE, 8), blk),
        ],
        out_shape=[
            jax.ShapeDtypeStruct((E, D), jnp.float32),
            jax.ShapeDtypeStruct((E, D), jnp.float32),
            jax.ShapeDtypeStruct((E, 8), jnp.float32),
        ],
    )(edge_attr, m_eo, m_s, m_r, We0, w1c, w3b, W2m, consts)


def kernel(node_attr, edge_attr, global_attr, edge_index, node_batch,
           edge_batch, num_nodes, num_edges, W_edge, b_edge, W_node, b_node,
           w1, b1, w2, w3, b3):
    N, D = node_attr.shape
    E = edge_attr.shape[0]
    H, HD = w2.shape
    row = edge_index[0]
    col = edge_index[1]

    A_e = node_attr @ W_edge[D:2 * D]
    B_e = edge_attr[:N] @ W_edge[2 * D:3 * D]
    c0 = global_attr @ W_edge[3 * D:] + b_edge
    A1 = node_attr @ w1[:D]
    B1 = node_attr @ w1[D:2 * D]
    Dv = node_attr @ w3[:D]
    cn = global_attr @ W_node[3 * D:] + b_node

    W2m = jnp.zeros((D, 8), jnp.float32)
    for h in range(H):
        W2m = W2m.at[h * HD:(h + 1) * HD, h].set(w2[h])

    consts = jnp.zeros((8, D), jnp.float32)
    consts = consts.at[0].set(c0[0]).at[1].set(b1)

    m_eo = A_e[row] + B_e[col]
    m_s = A1[row] + B1[col]
    m_r = B1[row] + A1[col]
    dv_c = Dv[col]
    dv_r = Dv[row]

    edge_out, F, lg = _edge_dense(edge_attr, m_eo, m_s, m_r, W_edge[:D],
                                  w1[2 * D:], w3[D:], W2m, consts)
    ls = lg[:, 0:2]
    lr = lg[:, 2:4]

    partials = _sc_segmax(lg, row, col, N)
    maxes = jnp.max(partials, axis=0)
    max_s = maxes[0:2].T
    max_r = maxes[2:4].T

    ex_s = jnp.exp(ls - max_s[row])
    ex_r = jnp.exp(lr - max_r[col])

    v_s = (dv_c + F + b3[None, :]).reshape(E, H, HD)
    v_r = (dv_r + F + b3[None, :]).reshape(E, H, HD)
    u_s = (ex_s[:, :, None] * v_s).reshape(E, D)
    u_r = (ex_r[:, :, None] * v_r).reshape(E, D)

    sent_raw = jax.ops.segment_sum(u_s, row, N)
    recv_raw = jax.ops.segment_sum(u_r, col, N)
    den_s = jax.ops.segment_sum(ex_s, row, N)
    den_r = jax.ops.segment_sum(ex_r, col, N)

    sent = (sent_raw.reshape(N, H, HD) /
            (den_s[:, :, None] + 1e-16)).reshape(N, D)
    recv = (recv_raw.reshape(N, H, HD) /
            (den_r[:, :, None] + 1e-16)).reshape(N, D)

    node_out = (node_attr @ W_node[:D] + sent @ W_node[D:2 * D] +
                recv @ W_node[2 * D:3 * D] + cn)
    return (node_out, edge_out, global_attr)


# speedup vs baseline: 1.0297x; 1.0098x over previous
import jax
import jax.numpy as jnp
from jax import lax
from jax.experimental import pallas as pl
from jax.experimental.pallas import tpu as pltpu
from jax.experimental.pallas import tpu_sc as plsc

_TE = 2000
_CH = 128
_NEG = -3.0e38


def _segmax_body(lg_hbm, row_hbm, col_hbm, out_hbm,
                 lgbuf, idxbuf, lgtail, idxtail, mt0, mt1, mt2, mt3):
    E = lg_hbm.shape[0] // 8
    N = mt0.shape[0]
    c = lax.axis_index("c")
    s = lax.axis_index("s")
    w = s * 2 + c
    mts = (mt0, mt1, mt2, mt3)

    def _init(i, _):
        mt0[pl.ds(i * 16, 16)] = jnp.full((16,), _NEG, jnp.float32)
        mt1[pl.ds(i * 16, 16)] = jnp.full((16,), _NEG, jnp.float32)
        mt2[pl.ds(i * 16, 16)] = jnp.full((16,), _NEG, jnp.float32)
        mt3[pl.ds(i * 16, 16)] = jnp.full((16,), _NEG, jnp.float32)
        return 0
    lax.fori_loop(0, N // 16, _init, 0)

    lanes = lax.iota(jnp.int32, 16)

    def _scatter_max(mt, idxv, val):
        cur = plsc.load_gather(mt, [idxv])
        tgt = jnp.maximum(cur, val)
        plsc.store_scatter(mt, [idxv], tgt)

        def cond(st):
            return st[0]

        def body(st):
            _, t = st
            cur2 = plsc.load_gather(mt, [idxv])
            need = cur2 < t
            plsc.store_scatter(mt, [idxv], jnp.maximum(cur2, t), mask=need)
            cur3 = plsc.load_gather(mt, [idxv])
            return (jnp.any(cur3 < t), t)

        first = plsc.load_gather(mt, [idxv])
        lax.while_loop(cond, body, (jnp.any(first < tgt), tgt))

    def _fields(lbuf, ibuf, g, fs):
        ids8 = (lanes + g * 16) * 8
        idxv = ibuf[pl.ds(g * 16, 16)]
        for f in fs:
            val = plsc.load_gather(lbuf, [ids8 + f])
            _scatter_max(mts[f], idxv, val)

    per_tile = (E // 32) // 16 * 16
    n_chunks = per_tile // _CH

    def _chunk(k, _):
        base = w * per_tile + k * _CH

        def _grp_row(g, _):
            _fields(lgbuf, idxbuf, g, (0, 1))
            return 0

        def _grp_col(g, _):
            _fields(lgbuf, idxbuf, g, (2, 3))
            return 0

        pltpu.sync_copy(lg_hbm.at[pl.ds(base * 8, _CH * 8)], lgbuf)
        pltpu.sync_copy(row_hbm.at[pl.ds(base, _CH)], idxbuf)
        lax.fori_loop(0, _CH // 16, _grp_row, 0)
        pltpu.sync_copy(col_hbm.at[pl.ds(base, _CH)], idxbuf)
        lax.fori_loop(0, _CH // 16, _grp_col, 0)
        return 0
    lax.fori_loop(0, n_chunks, _chunk, 0)

    rem = E - 32 * per_tile
    n_rem_groups = rem // 16
    if rem:
        @pl.when(w < n_rem_groups)
        def _():
            base = 32 * per_tile + w * 16
            pltpu.sync_copy(lg_hbm.at[pl.ds(base * 8, 128)], lgtail)
            pltpu.sync_copy(row_hbm.at[pl.ds(base, 16)], idxtail)
            _fields(lgtail, idxtail, 0, (0, 1))
            pltpu.sync_copy(col_hbm.at[pl.ds(base, 16)], idxtail)
            _fields(lgtail, idxtail, 0, (2, 3))

    pltpu.sync_copy(mt0, out_hbm.at[w, 0])
    pltpu.sync_copy(mt1, out_hbm.at[w, 1])
    pltpu.sync_copy(mt2, out_hbm.at[w, 2])
    pltpu.sync_copy(mt3, out_hbm.at[w, 3])


def _sc_segmax(lg, row, col, N):
    mesh = plsc.VectorSubcoreMesh(core_axis_name="c", subcore_axis_name="s")
    f = pl.kernel(
        _segmax_body,
        out_type=jax.ShapeDtypeStruct((32, 4, N), jnp.float32),
        mesh=mesh,
        compiler_params=pltpu.CompilerParams(needs_layout_passes=False),
        scratch_types=[
            pltpu.VMEM((_CH * 8,), jnp.float32),
            pltpu.VMEM((_CH,), jnp.int32),
            pltpu.VMEM((128,), jnp.float32),
            pltpu.VMEM((16,), jnp.int32),
            pltpu.VMEM((N,), jnp.float32),
            pltpu.VMEM((N,), jnp.float32),
            pltpu.VMEM((N,), jnp.float32),
            pltpu.VMEM((N,), jnp.float32),
        ],
    )
    return f(lg.reshape(-1), row, col)


def _edge_dense_body(ea_ref, meo_ref, ms_ref, mr_ref, we0_ref, w1c_ref,
                     w3b_ref, w2m_ref, consts_ref, eo_ref, f_ref, lg_ref):
    c0 = consts_ref[0, :]
    b1 = consts_ref[1, :]
    ea = ea_ref[...]
    eo = jnp.dot(ea, we0_ref[...], preferred_element_type=jnp.float32)
    eo = eo + meo_ref[...] + c0[None, :]
    eo_ref[...] = eo
    C = jnp.dot(eo, w1c_ref[...], preferred_element_type=jnp.float32)
    f_ref[...] = jnp.dot(eo, w3b_ref[...], preferred_element_type=jnp.float32)
    xs = ms_ref[...] + C + b1[None, :]
    xr = mr_ref[...] + C + b1[None, :]
    xs = jnp.where(xs > 0, xs, 0.01 * xs)
    xr = jnp.where(xr > 0, xr, 0.01 * xr)
    w2m = w2m_ref[...]
    ls = jnp.dot(xs, w2m, preferred_element_type=jnp.float32)
    lr = jnp.dot(xr, w2m, preferred_element_type=jnp.float32)
    lg_ref[...] = jnp.concatenate([ls[:, :2], lr[:, :2], ls[:, :2], lr[:, :2]],
                                  axis=1)


def _edge_dense(edge_attr, m_eo, m_s, m_r, We0, w1c, w3b, W2m, consts):
    E, D = edge_attr.shape
    grid = E // _TE
    blk = lambda i: (i, 0)
    whole = lambda i: (0, 0)
    return pl.pallas_call(
        _edge_dense_body,
        grid=(grid,),
        in_specs=[
            pl.BlockSpec((_TE, D), blk),
            pl.BlockSpec((_TE, D), blk),
            pl.BlockSpec((_TE, D), blk),
            pl.BlockSpec((_TE, D), blk),
            pl.BlockSpec((D, D), whole),
            pl.BlockSpec((D, D), whole),
            pl.BlockSpec((D, D), whole),
            pl.BlockSpec((D, 8), whole),
            pl.BlockSpec((8, D), whole),
        ],
        out_specs=[
            pl.BlockSpec((_TE, D), blk),
            pl.BlockSpec((_TE, D), blk),
            pl.BlockSpec((_TE, 8), blk),
        ],
        out_shape=[
            jax.ShapeDtypeStruct((E, D), jnp.float32),
            jax.ShapeDtypeStruct((E, D), jnp.float32),
            jax.ShapeDtypeStruct((E, 8), jnp.float32),
        ],
    )(edge_attr, m_eo, m_s, m_r, We0, w1c, w3b, W2m, consts)


def _node_dense_body(na_ref, sr_ref, rr_ref, ds_ref, dr_ref, wn0_ref,
                     wn1_ref, wn2_ref, consts_ref, out_ref):
    cn = consts_ref[2, :]
    sent = sr_ref[...] / (ds_ref[...] + 1e-16)
    recv = rr_ref[...] / (dr_ref[...] + 1e-16)
    out = jnp.dot(na_ref[...], wn0_ref[...],
                  preferred_element_type=jnp.float32)
    out += jnp.dot(sent, wn1_ref[...], preferred_element_type=jnp.float32)
    out += jnp.dot(recv, wn2_ref[...], preferred_element_type=jnp.float32)
    out_ref[...] = out + cn[None, :]


def _node_dense(node_attr, sent_raw, recv_raw, den_sb, den_rb,
                Wn0, Wn1, Wn2, consts):
    N, D = node_attr.shape
    TN = 2000
    blk = lambda i: (i, 0)
    whole = lambda i: (0, 0)
    return pl.pallas_call(
        _node_dense_body,
        grid=(N // TN,),
        in_specs=[
            pl.BlockSpec((TN, D), blk),
            pl.BlockSpec((TN, D), blk),
            pl.BlockSpec((TN, D), blk),
            pl.BlockSpec((TN, D), blk),
            pl.BlockSpec((TN, D), blk),
            pl.BlockSpec((D, D), whole),
            pl.BlockSpec((D, D), whole),
            pl.BlockSpec((D, D), whole),
            pl.BlockSpec((8, D), whole),
        ],
        out_specs=pl.BlockSpec((TN, D), blk),
        out_shape=jax.ShapeDtypeStruct((N, D), jnp.float32),
    )(node_attr, sent_raw, recv_raw, den_sb, den_rb, Wn0, Wn1, Wn2, consts)


def kernel(node_attr, edge_attr, global_attr, edge_index, node_batch,
           edge_batch, num_nodes, num_edges, W_edge, b_edge, W_node, b_node,
           w1, b1, w2, w3, b3):
    N, D = node_attr.shape
    E = edge_attr.shape[0]
    H, HD = w2.shape
    row = edge_index[0]
    col = edge_index[1]

    A_e = node_attr @ W_edge[D:2 * D]
    B_e = edge_attr[:N] @ W_edge[2 * D:3 * D]
    c0 = global_attr @ W_edge[3 * D:] + b_edge
    A1 = node_attr @ w1[:D]
    B1 = node_attr @ w1[D:2 * D]
    Dv = node_attr @ w3[:D]
    cn = global_attr @ W_node[3 * D:] + b_node

    W2m = jnp.zeros((D, 8), jnp.float32)
    for h in range(H):
        W2m = W2m.at[h * HD:(h + 1) * HD, h].set(w2[h])

    consts = jnp.zeros((8, D), jnp.float32)
    consts = consts.at[0].set(c0[0]).at[1].set(b1).at[2].set(cn[0])

    m_eo = A_e[row] + B_e[col]
    m_s = A1[row] + B1[col]
    m_r = B1[row] + A1[col]
    dv_c = Dv[col]
    dv_r = Dv[row]

    edge_out, F, lg = _edge_dense(edge_attr, m_eo, m_s, m_r, W_edge[:D],
                                  w1[2 * D:], w3[D:], W2m, consts)
    ls = lg[:, 0:2]
    lr = lg[:, 2:4]

    partials = _sc_segmax(lg, row, col, N)
    maxes = jnp.max(partials, axis=0)
    max_s = maxes[0:2].T
    max_r = maxes[2:4].T

    ex_s = jnp.exp(ls - max_s[row])
    ex_r = jnp.exp(lr - max_r[col])

    v_s = (dv_c + F + b3[None, :]).reshape(E, H, HD)
    v_r = (dv_r + F + b3[None, :]).reshape(E, H, HD)
    u_s = (ex_s[:, :, None] * v_s).reshape(E, D)
    u_r = (ex_r[:, :, None] * v_r).reshape(E, D)

    sent_raw = jax.ops.segment_sum(u_s, row, N)
    recv_raw = jax.ops.segment_sum(u_r, col, N)
    den_s = jax.ops.segment_sum(ex_s, row, N)
    den_r = jax.ops.segment_sum(ex_r, col, N)

    den_sb = jnp.repeat(den_s, HD, axis=1)
    den_rb = jnp.repeat(den_r, HD, axis=1)
    node_out = _node_dense(node_attr, sent_raw, recv_raw, den_sb, den_rb,
                           W_node[:D], W_node[D:2 * D], W_node[2 * D:3 * D],
                           consts)
    return (node_out, edge_out, global_attr)


# speedup vs baseline: 1.0304x; 1.0007x over previous
import jax
import jax.numpy as jnp
from jax import lax
from jax.experimental import pallas as pl
from jax.experimental.pallas import tpu as pltpu
from jax.experimental.pallas import tpu_sc as plsc

_TE = 2000
_CH = 128
_NEG = -3.0e38


def _segmax_body(lg_hbm, row_hbm, col_hbm, out_hbm,
                 lgbuf, idxbuf, lgtail, idxtail, mt0, mt1, mt2, mt3):
    E = lg_hbm.shape[0] // 8
    N = mt0.shape[0]
    c = lax.axis_index("c")
    s = lax.axis_index("s")
    w = s * 2 + c
    mts = (mt0, mt1, mt2, mt3)

    def _init(i, _):
        mt0[pl.ds(i * 16, 16)] = jnp.full((16,), _NEG, jnp.float32)
        mt1[pl.ds(i * 16, 16)] = jnp.full((16,), _NEG, jnp.float32)
        mt2[pl.ds(i * 16, 16)] = jnp.full((16,), _NEG, jnp.float32)
        mt3[pl.ds(i * 16, 16)] = jnp.full((16,), _NEG, jnp.float32)
        return 0
    lax.fori_loop(0, N // 16, _init, 0)

    lanes = lax.iota(jnp.int32, 16)

    def _scatter_max(mt, idxv, val):
        cur = plsc.load_gather(mt, [idxv])
        tgt = jnp.maximum(cur, val)
        plsc.store_scatter(mt, [idxv], tgt)

        def cond(st):
            return st[0]

        def body(st):
            _, t = st
            cur2 = plsc.load_gather(mt, [idxv])
            need = cur2 < t
            plsc.store_scatter(mt, [idxv], jnp.maximum(cur2, t), mask=need)
            cur3 = plsc.load_gather(mt, [idxv])
            return (jnp.any(cur3 < t), t)

        first = plsc.load_gather(mt, [idxv])
        lax.while_loop(cond, body, (jnp.any(first < tgt), tgt))

    def _fields(lbuf, ibuf, g, fs):
        ids8 = (lanes + g * 16) * 8
        idxv = ibuf[pl.ds(g * 16, 16)]
        for f in fs:
            val = plsc.load_gather(lbuf, [ids8 + f])
            _scatter_max(mts[f], idxv, val)

    per_tile = (E // 32) // 16 * 16
    n_chunks = per_tile // _CH

    def _chunk(k, _):
        base = w * per_tile + k * _CH

        def _grp_row(g, _):
            _fields(lgbuf, idxbuf, g, (0, 1))
            return 0

        def _grp_col(g, _):
            _fields(lgbuf, idxbuf, g, (2, 3))
            return 0

        pltpu.sync_copy(lg_hbm.at[pl.ds(base * 8, _CH * 8)], lgbuf)
        pltpu.sync_copy(row_hbm.at[pl.ds(base, _CH)], idxbuf)
        lax.fori_loop(0, _CH // 16, _grp_row, 0)
        pltpu.sync_copy(col_hbm.at[pl.ds(base, _CH)], idxbuf)
        lax.fori_loop(0, _CH // 16, _grp_col, 0)
        return 0
    lax.fori_loop(0, n_chunks, _chunk, 0)

    rem = E - 32 * per_tile
    n_rem_groups = rem // 16
    if rem:
        @pl.when(w < n_rem_groups)
        def _():
            base = 32 * per_tile + w * 16
            pltpu.sync_copy(lg_hbm.at[pl.ds(base * 8, 128)], lgtail)
            pltpu.sync_copy(row_hbm.at[pl.ds(base, 16)], idxtail)
            _fields(lgtail, idxtail, 0, (0, 1))
            pltpu.sync_copy(col_hbm.at[pl.ds(base, 16)], idxtail)
            _fields(lgtail, idxtail, 0, (2, 3))

    pltpu.sync_copy(mt0, out_hbm.at[w, 0])
    pltpu.sync_copy(mt1, out_hbm.at[w, 1])
    pltpu.sync_copy(mt2, out_hbm.at[w, 2])
    pltpu.sync_copy(mt3, out_hbm.at[w, 3])


def _sc_segmax(lg, row, col, N):
    mesh = plsc.VectorSubcoreMesh(core_axis_name="c", subcore_axis_name="s")
    f = pl.kernel(
        _segmax_body,
        out_type=jax.ShapeDtypeStruct((32, 4, N), jnp.float32),
        mesh=mesh,
        compiler_params=pltpu.CompilerParams(needs_layout_passes=False),
        scratch_types=[
            pltpu.VMEM((_CH * 8,), jnp.float32),
            pltpu.VMEM((_CH,), jnp.int32),
            pltpu.VMEM((128,), jnp.float32),
            pltpu.VMEM((16,), jnp.int32),
            pltpu.VMEM((N,), jnp.float32),
            pltpu.VMEM((N,), jnp.float32),
            pltpu.VMEM((N,), jnp.float32),
            pltpu.VMEM((N,), jnp.float32),
        ],
    )
    return f(lg.reshape(-1), row, col)


def _edge_dense_body(ea_ref, meo_ref, ms_ref, mr_ref, we0_ref, w1c_ref,
                     w3b_ref, w2m_ref, consts_ref, eo_ref, f_ref, lg_ref):
    c0 = consts_ref[0, :]
    b1 = consts_ref[1, :]
    ea = ea_ref[...]
    eo = jnp.dot(ea, we0_ref[...], preferred_element_type=jnp.float32)
    eo = eo + meo_ref[...] + c0[None, :]
    eo_ref[...] = eo
    C = jnp.dot(eo, w1c_ref[...], preferred_element_type=jnp.float32)
    f_ref[...] = jnp.dot(eo, w3b_ref[...], preferred_element_type=jnp.float32)
    xs = ms_ref[...] + C + b1[None, :]
    xr = mr_ref[...] + C + b1[None, :]
    xs = jnp.where(xs > 0, xs, 0.01 * xs)
    xr = jnp.where(xr > 0, xr, 0.01 * xr)
    w2m = w2m_ref[...]
    ls = jnp.dot(xs, w2m, preferred_element_type=jnp.float32)
    lr = jnp.dot(xr, w2m, preferred_element_type=jnp.float32)
    lg_ref[...] = jnp.concatenate([ls[:, :2], lr[:, :2], ls[:, :2], lr[:, :2]],
                                  axis=1)


def _edge_dense(edge_attr, m_eo, m_s, m_r, We0, w1c, w3b, W2m, consts):
    E, D = edge_attr.shape
    grid = E // _TE
    blk = lambda i: (i, 0)
    whole = lambda i: (0, 0)
    return pl.pallas_call(
        _edge_dense_body,
        grid=(grid,),
        in_specs=[
            pl.BlockSpec((_TE, D), blk),
            pl.BlockSpec((_TE, D), blk),
            pl.BlockSpec((_TE, D), blk),
            pl.BlockSpec((_TE, D), blk),
            pl.BlockSpec((D, D), whole),
            pl.BlockSpec((D, D), whole),
            pl.BlockSpec((D, D), whole),
            pl.BlockSpec((D, 8), whole),
            pl.BlockSpec((8, D), whole),
        ],
        out_specs=[
            pl.BlockSpec((_TE, D), blk),
            pl.BlockSpec((_TE, D), blk),
            pl.BlockSpec((_TE, 8), blk),
        ],
        out_shape=[
            jax.ShapeDtypeStruct((E, D), jnp.float32),
            jax.ShapeDtypeStruct((E, D), jnp.float32),
            jax.ShapeDtypeStruct((E, 8), jnp.float32),
        ],
    )(edge_attr, m_eo, m_s, m_r, We0, w1c, w3b, W2m, consts)


def _node_dense_body(na_ref, sr_ref, rr_ref, ds_ref, dr_ref, wn0_ref,
                     wn1_ref, wn2_ref, consts_ref, out_ref):
    cn = consts_ref[2, :]
    sent = sr_ref[...] / (ds_ref[...] + 1e-16)
    recv = rr_ref[...] / (dr_ref[...] + 1e-16)
    out = jnp.dot(na_ref[...], wn0_ref[...],
                  preferred_element_type=jnp.float32)
    out += jnp.dot(sent, wn1_ref[...], preferred_element_type=jnp.float32)
    out += jnp.dot(recv, wn2_ref[...], preferred_element_type=jnp.float32)
    out_ref[...] = out + cn[None, :]


def _node_dense(node_attr, sent_raw, recv_raw, den_sb, den_rb,
                Wn0, Wn1, Wn2, consts):
    N, D = node_attr.shape
    TN = 2000
    blk = lambda i: (i, 0)
    whole = lambda i: (0, 0)
    return pl.pallas_call(
        _node_dense_body,
        grid=(N // TN,),
        in_specs=[
            pl.BlockSpec((TN, D), blk),
            pl.BlockSpec((TN, D), blk),
            pl.BlockSpec((TN, D), blk),
            pl.BlockSpec((TN, D), blk),
            pl.BlockSpec((TN, D), blk),
            pl.BlockSpec((D, D), whole),
            pl.BlockSpec((D, D), whole),
            pl.BlockSpec((D, D), whole),
            pl.BlockSpec((8, D), whole),
        ],
        out_specs=pl.BlockSpec((TN, D), blk),
        out_shape=jax.ShapeDtypeStruct((N, D), jnp.float32),
    )(node_attr, sent_raw, recv_raw, den_sb, den_rb, Wn0, Wn1, Wn2, consts)


def kernel(node_attr, edge_attr, global_attr, edge_index, node_batch,
           edge_batch, num_nodes, num_edges, W_edge, b_edge, W_node, b_node,
           w1, b1, w2, w3, b3):
    N, D = node_attr.shape
    E = edge_attr.shape[0]
    H, HD = w2.shape
    row = edge_index[0]
    col = edge_index[1]

    A_e = node_attr @ W_edge[D:2 * D]
    B_e = edge_attr[:N] @ W_edge[2 * D:3 * D]
    c0 = global_attr @ W_edge[3 * D:] + b_edge
    A1 = node_attr @ w1[:D]
    B1 = node_attr @ w1[D:2 * D]
    Dv = node_attr @ w3[:D]
    cn = global_attr @ W_node[3 * D:] + b_node

    W2m = jnp.zeros((D, 8), jnp.float32)
    for h in range(H):
        W2m = W2m.at[h * HD:(h + 1) * HD, h].set(w2[h])

    consts = jnp.zeros((8, D), jnp.float32)
    consts = consts.at[0].set(c0[0]).at[1].set(b1).at[2].set(cn[0])

    m_eo = A_e[row] + B_e[col]
    m_s = A1[row] + B1[col]
    m_r = B1[row] + A1[col]
    dv_c = Dv[col]
    dv_r = Dv[row]

    edge_out, F, lg = _edge_dense(edge_attr, m_eo, m_s, m_r, W_edge[:D],
                                  w1[2 * D:], w3[D:], W2m, consts)
    ls = lg[:, 0:2]
    lr = lg[:, 2:4]

    partials = _sc_segmax(lg, row, col, N)
    maxes = jnp.max(partials, axis=0)
    max_s = maxes[0:2].T
    max_r = maxes[2:4].T

    ex_s = jnp.exp(ls - max_s[row])
    ex_r = jnp.exp(lr - max_r[col])

    v_s = (dv_c + F + b3[None, :]).reshape(E, H, HD)
    v_r = (dv_r + F + b3[None, :]).reshape(E, H, HD)
    u_s = (ex_s[:, :, None] * v_s).reshape(E, D)
    u_r = (ex_r[:, :, None] * v_r).reshape(E, D)

    us_ext = jnp.concatenate([u_s, ex_s], axis=1)
    ur_ext = jnp.concatenate([u_r, ex_r], axis=1)
    sent_ext = jax.ops.segment_sum(us_ext, row, N)
    recv_ext = jax.ops.segment_sum(ur_ext, col, N)
    sent_raw = sent_ext[:, :D]
    recv_raw = recv_ext[:, :D]
    den_s = sent_ext[:, D:]
    den_r = recv_ext[:, D:]

    den_sb = jnp.repeat(den_s, HD, axis=1)
    den_rb = jnp.repeat(den_r, HD, axis=1)
    node_out = _node_dense(node_attr, sent_raw, recv_raw, den_sb, den_rb,
                           W_node[:D], W_node[D:2 * D], W_node[2 * D:3 * D],
                           consts)
    return (node_out, edge_out, global_attr)
